# baseline (speedup 1.0000x reference)
import functools

import jax
import jax.numpy as jnp
from jax import lax
from jax.experimental import pallas as pl
from jax.experimental.pallas import tpu as pltpu
from jax.experimental.pallas import tpu_sc as plsc

N_NODES = 1_000_000
N_EDGES = 1_600_000
N_POS = 1024
PI = 0.25

NC, NS, L = 2, 16, 16
NW = NC * NS
NPAD = 1_007_616
SLICE = NPAD // NS
ZCH = SLICE // 16
NZC = 16
ROWS = NPAD // 128
ECH = N_EDGES // 128
RPT = 390
REM = ECH - NW * RPT
BLK = 130
NBLK = RPT // BLK
PPT = N_POS // NW


def _sc_body(idx_hbm, pos_hbm, inp_hbm, zeros_hbm, ones_hbm,
             mask0_out, mask1_out, inpos_out,
             idx0_v, idx1_v, idx2_v, ones_v, zstage_v, zstage2_v, pidx_v,
             pip_v, pzidx_v, gsem, ssem, osem, csem, shared):
    cid = lax.axis_index("c")
    sid = lax.axis_index("s")
    wid = sid * NC + cid
    bufs = [idx0_v, idx1_v, idx2_v]

    def stage(b):
        return pltpu.make_async_copy(
            idx_hbm.at[pl.ds(wid * RPT + b * BLK, BLK), 0], bufs[b], ssem)

    for b in range(NBLK):
        stage(b).start()
    pltpu.sync_copy(ones_hbm, ones_v)

    pltpu.sync_copy(zeros_hbm, zstage_v)

    def zchunk(k):
        return pltpu.make_async_copy(
            zstage_v, shared.at[pl.ds(sid * SLICE + k * ZCH, ZCH)], osem)

    for k in range(NZC):
        zchunk(k).start()
    for k in range(NZC):
        zchunk(k).wait()

    pr = wid // 4
    pc = (wid % 4) * PPT
    pltpu.sync_copy(pos_hbm.at[pr, pl.ds(pc, PPT)], pidx_v)
    pltpu.async_copy(inp_hbm.at[pidx_v], pip_v, gsem).wait()
    pltpu.sync_copy(pip_v, inpos_out.at[pl.ds(wid * PPT, PPT)])

    plsc.subcore_barrier()

    for b in range(NBLK):
        stage(b).wait()
        cur = bufs[b]

        def fire(j, carry):
            pltpu.make_async_copy(ones_v, shared.at[cur.at[j]], csem).start(add=True)
            return carry

        lax.fori_loop(0, BLK, fire, 0)

    def drain(j, carry):
        pltpu.make_async_copy(ones_v, shared.at[idx0_v.at[0]], csem).wait()
        return carry

    lax.fori_loop(0, NBLK * BLK, drain, 0)

    @pl.when(wid < REM)
    def _():
        pltpu.sync_copy(idx_hbm.at[NW * RPT + wid, 0], pzidx_v)
        pltpu.sync_copy(ones_v, shared.at[pzidx_v], add=True)

    plsc.subcore_barrier()

    @pl.when(sid < 8)
    def _():
        pltpu.sync_copy(pos_hbm.at[sid], pzidx_v)
        pltpu.sync_copy(zstage_v.at[pl.ds(0, 128)], shared.at[pzidx_v])

    plsc.subcore_barrier()

    stages = [zstage_v, zstage2_v]

    def out_copy(k, buf):
        sl = pl.ds(sid * SLICE + k * ZCH, ZCH)

        @pl.when(cid == 0)
        def _():
            pltpu.make_async_copy(buf, mask0_out.at[sl], osem).start()

        @pl.when(cid == 1)
        def _():
            pltpu.make_async_copy(buf, mask1_out.at[sl], osem).start()

    def out_wait(k, buf):
        sl = pl.ds(sid * SLICE + k * ZCH, ZCH)

        @pl.when(cid == 0)
        def _():
            pltpu.make_async_copy(buf, mask0_out.at[sl], osem).wait()

        @pl.when(cid == 1)
        def _():
            pltpu.make_async_copy(buf, mask1_out.at[sl], osem).wait()

    for k in range(NZC):
        buf = stages[k % 2]
        if k >= 2:
            out_wait(k - 2, buf)
        pltpu.sync_copy(shared.at[pl.ds(sid * SLICE + k * ZCH, ZCH)], buf)
        out_copy(k, buf)
    out_wait(NZC - 2, stages[0])
    out_wait(NZC - 1, stages[1])


_sc_call = functools.partial(
    pl.kernel,
    out_type=[
        jax.ShapeDtypeStruct((NPAD,), jnp.float32),
        jax.ShapeDtypeStruct((NPAD,), jnp.float32),
        jax.ShapeDtypeStruct((N_POS,), jnp.float32),
    ],
    mesh=plsc.VectorSubcoreMesh(core_axis_name="c", subcore_axis_name="s"),
    scratch_types=[
        pltpu.VMEM((BLK, 128), jnp.int32),
        pltpu.VMEM((BLK, 128), jnp.int32),
        pltpu.VMEM((BLK, 128), jnp.int32),
        pltpu.VMEM((128,), jnp.float32),
        pltpu.VMEM((ZCH,), jnp.float32),
        pltpu.VMEM((ZCH,), jnp.float32),
        pltpu.VMEM((PPT,), jnp.int32),
        pltpu.VMEM((PPT,), jnp.float32),
        pltpu.VMEM((128,), jnp.int32),
        pltpu.SemaphoreType.DMA,
        pltpu.SemaphoreType.DMA,
        pltpu.SemaphoreType.DMA,
        pltpu.SemaphoreType.DMA,
        pltpu.VMEM_SHARED((NPAD,), jnp.float32),
    ],
)(_sc_body)


G = 2


def _sp_body(x_ref, sp_ref):
    x = x_ref[...]
    sp_ref[...] = jnp.maximum(x, 0.0) + jnp.log1p(jnp.exp(-jnp.abs(x)))


def _tc_body(sp_ref, m0_ref, m1_ref, ip_ref, o_ref, acc_ref):
    i = pl.program_id(0)

    @pl.when(i == 0)
    def _():
        acc_ref[0] = 0.0
        acc_ref[1] = 0.0

    neg = (m0_ref[...] + m1_ref[...]) > 0.0
    acc_ref[0] += jnp.sum(jnp.where(neg, sp_ref[...], 0.0))
    acc_ref[1] += jnp.sum(neg.astype(jnp.float32))

    @pl.when(i == G - 1)
    def _():
        ip = ip_ref[...]
        sp_p = jnp.maximum(ip, 0.0) + jnp.log1p(jnp.exp(-jnp.abs(ip)))
        rp_plus = jnp.mean(sp_p - ip)
        rp_minus = jnp.mean(sp_p)
        loss_u = jnp.maximum(acc_ref[0] / acc_ref[1] - PI * rp_minus, 0.0)
        o_ref[0, 0] = PI * rp_plus + loss_u


def kernel(input, nodes, pos_labels):
    idx3 = nodes.astype(jnp.int32).reshape(2, ECH, 128).transpose(1, 0, 2)
    pos2d = pos_labels.astype(jnp.int32).reshape(N_POS // 128, 128)
    zeros_in = jnp.zeros((ZCH,), jnp.float32)
    ones_in = jnp.ones((128,), jnp.float32)

    mask0, mask1, in_pos = _sc_call(idx3, pos2d, input, zeros_in, ones_in)

    x_pad = jnp.concatenate(
        [input, jnp.zeros((NPAD - N_NODES,), jnp.float32)]
    ).reshape(ROWS, 128)

    rb = ROWS // G
    sp = pl.pallas_call(
        _sp_body,
        grid=(G,),
        in_specs=[pl.BlockSpec((rb, 128), lambda i: (i, 0))],
        out_shape=jax.ShapeDtypeStruct((ROWS, 128), jnp.float32),
        out_specs=pl.BlockSpec((rb, 128), lambda i: (i, 0)),
    )(x_pad)

    out = pl.pallas_call(
        _tc_body,
        grid=(G,),
        in_specs=[
            pl.BlockSpec((rb, 128), lambda i: (i, 0)),
            pl.BlockSpec((rb, 128), lambda i: (i, 0)),
            pl.BlockSpec((rb, 128), lambda i: (i, 0)),
            pl.BlockSpec((N_POS // 128, 128), lambda i: (0, 0)),
        ],
        out_shape=jax.ShapeDtypeStruct((1, 1), jnp.float32),
        out_specs=pl.BlockSpec(memory_space=pltpu.SMEM),
        scratch_shapes=[pltpu.SMEM((2,), jnp.float32)],
    )(sp, mask0.reshape(ROWS, 128), mask1.reshape(ROWS, 128),
      in_pos.reshape(N_POS // 128, 128))
    return out[0, 0]


# speedup vs baseline: 1.0469x; 1.0469x over previous
import functools

import jax
import jax.numpy as jnp
from jax import lax
from jax.experimental import pallas as pl
from jax.experimental.pallas import tpu as pltpu
from jax.experimental.pallas import tpu_sc as plsc

N_NODES = 1_000_000
N_EDGES = 1_600_000
N_POS = 1024
PI = 0.25

NC, NS, L = 2, 16, 16
NW = NC * NS
NPAD = 1_007_616
SLICE = NPAD // NS
ZCH = SLICE // 8
NZC = 8
ROWS = NPAD // 128
ECH = N_EDGES // 128
RPT = 390
REM = ECH - NW * RPT
BLK = 130
NBLK = RPT // BLK
PPT = N_POS // NW


def _sc_body(idx_hbm, pos_hbm, inp_hbm, zeros_hbm, ones_hbm,
             mask0_out, mask1_out, inpos_out,
             idx0_v, idx1_v, ones_v, zstage_v, zstage2_v, pidx_v,
             pip_v, pzidx_v, gsem, ssem, osem, csem, shared):
    cid = lax.axis_index("c")
    sid = lax.axis_index("s")
    wid = sid * NC + cid
    bufs = [idx0_v, idx1_v]

    def stage(b):
        return pltpu.make_async_copy(
            idx_hbm.at[pl.ds(wid * RPT + b * BLK, BLK), 0], bufs[b % 2], ssem)

    stage(0).start()
    pltpu.sync_copy(ones_hbm, ones_v)

    pltpu.sync_copy(zeros_hbm, zstage_v)

    def zchunk(k):
        return pltpu.make_async_copy(
            zstage_v, shared.at[pl.ds(sid * SLICE + k * ZCH, ZCH)], osem)

    for k in range(NZC):
        zchunk(k).start()
    for k in range(NZC):
        zchunk(k).wait()

    pr = wid // 4
    pc = (wid % 4) * PPT
    pltpu.sync_copy(pos_hbm.at[pr, pl.ds(pc, PPT)], pidx_v)
    pltpu.async_copy(inp_hbm.at[pidx_v], pip_v, gsem).wait()
    pltpu.sync_copy(pip_v, inpos_out.at[pl.ds(wid * PPT, PPT)])

    plsc.subcore_barrier()

    for b in range(NBLK):
        stage(b).wait()
        if b + 1 < NBLK:
            stage(b + 1).start()
        cur = bufs[b % 2]

        def fire(j, carry):
            pltpu.make_async_copy(ones_v, shared.at[cur.at[j]], csem).start(add=True)
            return carry

        def drain(j, carry):
            pltpu.make_async_copy(ones_v, shared.at[cur.at[j]], csem).wait()
            return carry

        lax.fori_loop(0, BLK, fire, 0)
        lax.fori_loop(0, BLK, drain, 0)

    @pl.when(wid < REM)
    def _():
        pltpu.sync_copy(idx_hbm.at[NW * RPT + wid, 0], pzidx_v)
        pltpu.sync_copy(ones_v, shared.at[pzidx_v], add=True)

    plsc.subcore_barrier()

    @pl.when(sid < 8)
    def _():
        pltpu.sync_copy(pos_hbm.at[sid], pzidx_v)
        pltpu.sync_copy(zstage_v.at[pl.ds(0, 128)], shared.at[pzidx_v])

    plsc.subcore_barrier()

    stages = [zstage_v, zstage2_v]

    def out_copy(k, buf):
        sl = pl.ds(sid * SLICE + k * ZCH, ZCH)

        @pl.when(cid == 0)
        def _():
            pltpu.make_async_copy(buf, mask0_out.at[sl], osem).start()

        @pl.when(cid == 1)
        def _():
            pltpu.make_async_copy(buf, mask1_out.at[sl], osem).start()

    def out_wait(k, buf):
        sl = pl.ds(sid * SLICE + k * ZCH, ZCH)

        @pl.when(cid == 0)
        def _():
            pltpu.make_async_copy(buf, mask0_out.at[sl], osem).wait()

        @pl.when(cid == 1)
        def _():
            pltpu.make_async_copy(buf, mask1_out.at[sl], osem).wait()

    for k in range(NZC):
        buf = stages[k % 2]
        if k >= 2:
            out_wait(k - 2, buf)
        pltpu.sync_copy(shared.at[pl.ds(sid * SLICE + k * ZCH, ZCH)], buf)
        out_copy(k, buf)
    out_wait(NZC - 2, stages[0])
    out_wait(NZC - 1, stages[1])


_sc_call = functools.partial(
    pl.kernel,
    out_type=[
        jax.ShapeDtypeStruct((NPAD,), jnp.float32),
        jax.ShapeDtypeStruct((NPAD,), jnp.float32),
        jax.ShapeDtypeStruct((N_POS,), jnp.float32),
    ],
    mesh=plsc.VectorSubcoreMesh(core_axis_name="c", subcore_axis_name="s"),
    scratch_types=[
        pltpu.VMEM((BLK, 128), jnp.int32),
        pltpu.VMEM((BLK, 128), jnp.int32),
        pltpu.VMEM((128,), jnp.float32),
        pltpu.VMEM((ZCH,), jnp.float32),
        pltpu.VMEM((ZCH,), jnp.float32),
        pltpu.VMEM((PPT,), jnp.int32),
        pltpu.VMEM((PPT,), jnp.float32),
        pltpu.VMEM((128,), jnp.int32),
        pltpu.SemaphoreType.DMA,
        pltpu.SemaphoreType.DMA,
        pltpu.SemaphoreType.DMA,
        pltpu.SemaphoreType.DMA,
        pltpu.VMEM_SHARED((NPAD,), jnp.float32),
    ],
)(_sc_body)


G = 2


def _sp_body(x_ref, sp_ref):
    x = x_ref[...]
    sp_ref[...] = jnp.maximum(x, 0.0) + jnp.log1p(jnp.exp(-jnp.abs(x)))


def _tc_body(sp_ref, m0_ref, m1_ref, ip_ref, o_ref, acc_ref):
    i = pl.program_id(0)

    @pl.when(i == 0)
    def _():
        acc_ref[0] = 0.0
        acc_ref[1] = 0.0

    neg = (m0_ref[...] + m1_ref[...]) > 0.0
    acc_ref[0] += jnp.sum(jnp.where(neg, sp_ref[...], 0.0))
    acc_ref[1] += jnp.sum(neg.astype(jnp.float32))

    @pl.when(i == G - 1)
    def _():
        ip = ip_ref[...]
        sp_p = jnp.maximum(ip, 0.0) + jnp.log1p(jnp.exp(-jnp.abs(ip)))
        rp_plus = jnp.mean(sp_p - ip)
        rp_minus = jnp.mean(sp_p)
        loss_u = jnp.maximum(acc_ref[0] / acc_ref[1] - PI * rp_minus, 0.0)
        o_ref[0, 0] = PI * rp_plus + loss_u


def kernel(input, nodes, pos_labels):
    idx3 = nodes.astype(jnp.int32).reshape(2, ECH, 128).transpose(1, 0, 2)
    pos2d = pos_labels.astype(jnp.int32).reshape(N_POS // 128, 128)
    zeros_in = jnp.zeros((ZCH,), jnp.float32)
    ones_in = jnp.ones((128,), jnp.float32)

    mask0, mask1, in_pos = _sc_call(idx3, pos2d, input, zeros_in, ones_in)

    x_pad = jnp.concatenate(
        [input, jnp.zeros((NPAD - N_NODES,), jnp.float32)]
    ).reshape(ROWS, 128)

    rb = ROWS // G
    sp = pl.pallas_call(
        _sp_body,
        grid=(G,),
        in_specs=[pl.BlockSpec((rb, 128), lambda i: (i, 0))],
        out_shape=jax.ShapeDtypeStruct((ROWS, 128), jnp.float32),
        out_specs=pl.BlockSpec((rb, 128), lambda i: (i, 0)),
    )(x_pad)

    out = pl.pallas_call(
        _tc_body,
        grid=(G,),
        in_specs=[
            pl.BlockSpec((rb, 128), lambda i: (i, 0)),
            pl.BlockSpec((rb, 128), lambda i: (i, 0)),
            pl.BlockSpec((rb, 128), lambda i: (i, 0)),
            pl.BlockSpec((N_POS // 128, 128), lambda i: (0, 0)),
        ],
        out_shape=jax.ShapeDtypeStruct((1, 1), jnp.float32),
        out_specs=pl.BlockSpec(memory_space=pltpu.SMEM),
        scratch_shapes=[pltpu.SMEM((2,), jnp.float32)],
    )(sp, mask0.reshape(ROWS, 128), mask1.reshape(ROWS, 128),
      in_pos.reshape(N_POS // 128, 128))
    return out[0, 0]


# speedup vs baseline: 1.0540x; 1.0068x over previous
import functools

import jax
import jax.numpy as jnp
from jax import lax
from jax.experimental import pallas as pl
from jax.experimental.pallas import tpu as pltpu
from jax.experimental.pallas import tpu_sc as plsc

N_NODES = 1_000_000
N_EDGES = 1_600_000
N_POS = 1024
PI = 0.25

NC, NS, L = 2, 16, 16
NW = NC * NS
NPAD = 1_007_616
SLICE = NPAD // NS
ZCH = SLICE // 8
NZC = 8
ROWS = NPAD // 128
ECH = N_EDGES // 128
RPT = 390
REM = ECH - NW * RPT
BLK = 130
NBLK = RPT // BLK
PPT = N_POS // NW


def _sc_body(idx_hbm, pos_hbm, inp_hbm, zeros_hbm,
             mask0_out, mask1_out, inpos_out,
             idx0_v, idx1_v, ones_v, zstage_v, zstage2_v, pidx_v,
             pip_v, pzidx_v, gsem, ssem, osem, csem, shared):
    cid = lax.axis_index("c")
    sid = lax.axis_index("s")
    wid = sid * NC + cid
    bufs = [idx0_v, idx1_v]

    def stage(b):
        return pltpu.make_async_copy(
            idx_hbm.at[pl.ds(wid * RPT + b * BLK, BLK), 0], bufs[b % 2], ssem)

    stage(0).start()
    for i in range(8):
        ones_v[pl.ds(i * 16, 16)] = jnp.full((16,), 1.0, dtype=jnp.float32)

    pltpu.sync_copy(zeros_hbm, zstage_v)

    def zchunk(k):
        return pltpu.make_async_copy(
            zstage_v, shared.at[pl.ds(sid * SLICE + k * ZCH, ZCH)], osem)

    for k in range(NZC):
        zchunk(k).start()
    for k in range(NZC):
        zchunk(k).wait()

    pr = wid // 4
    pc = (wid % 4) * PPT
    pltpu.sync_copy(pos_hbm.at[pr, pl.ds(pc, PPT)], pidx_v)
    pltpu.async_copy(inp_hbm.at[pidx_v], pip_v, gsem).wait()
    pltpu.sync_copy(pip_v, inpos_out.at[pl.ds(wid * PPT, PPT)])

    plsc.subcore_barrier()

    for b in range(NBLK):
        stage(b).wait()
        if b + 1 < NBLK:
            stage(b + 1).start()
        cur = bufs[b % 2]

        def fire(j, carry):
            pltpu.make_async_copy(ones_v, shared.at[cur.at[j]], csem).start(add=True)
            return carry

        def drain(j, carry):
            pltpu.make_async_copy(ones_v, shared.at[cur.at[j]], csem).wait()
            return carry

        lax.fori_loop(0, BLK, fire, 0)
        lax.fori_loop(0, BLK, drain, 0)

    @pl.when(wid < REM)
    def _():
        pltpu.sync_copy(idx_hbm.at[NW * RPT + wid, 0], pzidx_v)
        pltpu.sync_copy(ones_v, shared.at[pzidx_v], add=True)

    plsc.subcore_barrier()

    @pl.when(sid < 8)
    def _():
        pltpu.sync_copy(pos_hbm.at[sid], pzidx_v)
        pltpu.sync_copy(zstage_v.at[pl.ds(0, 128)], shared.at[pzidx_v])

    plsc.subcore_barrier()

    stages = [zstage_v, zstage2_v]

    def out_copy(k, buf):
        sl = pl.ds(sid * SLICE + k * ZCH, ZCH)

        @pl.when(cid == 0)
        def _():
            pltpu.make_async_copy(buf, mask0_out.at[sl], osem).start()

        @pl.when(cid == 1)
        def _():
            pltpu.make_async_copy(buf, mask1_out.at[sl], osem).start()

    def out_wait(k, buf):
        sl = pl.ds(sid * SLICE + k * ZCH, ZCH)

        @pl.when(cid == 0)
        def _():
            pltpu.make_async_copy(buf, mask0_out.at[sl], osem).wait()

        @pl.when(cid == 1)
        def _():
            pltpu.make_async_copy(buf, mask1_out.at[sl], osem).wait()

    for k in range(NZC):
        buf = stages[k % 2]
        if k >= 2:
            out_wait(k - 2, buf)
        pltpu.sync_copy(shared.at[pl.ds(sid * SLICE + k * ZCH, ZCH)], buf)
        out_copy(k, buf)
    out_wait(NZC - 2, stages[0])
    out_wait(NZC - 1, stages[1])


_sc_call = functools.partial(
    pl.kernel,
    out_type=[
        jax.ShapeDtypeStruct((NPAD,), jnp.float32),
        jax.ShapeDtypeStruct((NPAD,), jnp.float32),
        jax.ShapeDtypeStruct((N_POS,), jnp.float32),
    ],
    mesh=plsc.VectorSubcoreMesh(core_axis_name="c", subcore_axis_name="s"),
    scratch_types=[
        pltpu.VMEM((BLK, 128), jnp.int32),
        pltpu.VMEM((BLK, 128), jnp.int32),
        pltpu.VMEM((128,), jnp.float32),
        pltpu.VMEM((ZCH,), jnp.float32),
        pltpu.VMEM((ZCH,), jnp.float32),
        pltpu.VMEM((PPT,), jnp.int32),
        pltpu.VMEM((PPT,), jnp.float32),
        pltpu.VMEM((128,), jnp.int32),
        pltpu.SemaphoreType.DMA,
        pltpu.SemaphoreType.DMA,
        pltpu.SemaphoreType.DMA,
        pltpu.SemaphoreType.DMA,
        pltpu.VMEM_SHARED((NPAD,), jnp.float32),
    ],
)(_sc_body)


G = 2


def _sp_body(x_ref, sp_ref):
    x = x_ref[...]
    sp_ref[...] = jnp.maximum(x, 0.0) + jnp.log1p(jnp.exp(-jnp.abs(x)))


def _tc_body(sp_ref, m0_ref, m1_ref, ip_ref, o_ref, acc_ref):
    i = pl.program_id(0)

    @pl.when(i == 0)
    def _():
        acc_ref[0] = 0.0
        acc_ref[1] = 0.0

    neg = (m0_ref[...] + m1_ref[...]) > 0.0
    acc_ref[0] += jnp.sum(jnp.where(neg, sp_ref[...], 0.0))
    acc_ref[1] += jnp.sum(neg.astype(jnp.float32))

    @pl.when(i == G - 1)
    def _():
        ip = ip_ref[...]
        sp_p = jnp.maximum(ip, 0.0) + jnp.log1p(jnp.exp(-jnp.abs(ip)))
        rp_plus = jnp.mean(sp_p - ip)
        rp_minus = jnp.mean(sp_p)
        loss_u = jnp.maximum(acc_ref[0] / acc_ref[1] - PI * rp_minus, 0.0)
        o_ref[0, 0] = PI * rp_plus + loss_u


def kernel(input, nodes, pos_labels):
    idx3 = nodes.astype(jnp.int32).reshape(2, ECH, 128).transpose(1, 0, 2)
    pos2d = pos_labels.astype(jnp.int32).reshape(N_POS // 128, 128)
    zeros_in = jnp.zeros((ZCH,), jnp.float32)

    mask0, mask1, in_pos = _sc_call(idx3, pos2d, input, zeros_in)

    x_pad = jnp.concatenate(
        [input, jnp.zeros((NPAD - N_NODES,), jnp.float32)]
    ).reshape(ROWS, 128)

    rb = ROWS // G
    sp = pl.pallas_call(
        _sp_body,
        grid=(G,),
        in_specs=[pl.BlockSpec((rb, 128), lambda i: (i, 0))],
        out_shape=jax.ShapeDtypeStruct((ROWS, 128), jnp.float32),
        out_specs=pl.BlockSpec((rb, 128), lambda i: (i, 0)),
    )(x_pad)

    out = pl.pallas_call(
        _tc_body,
        grid=(G,),
        in_specs=[
            pl.BlockSpec((rb, 128), lambda i: (i, 0)),
            pl.BlockSpec((rb, 128), lambda i: (i, 0)),
            pl.BlockSpec((rb, 128), lambda i: (i, 0)),
            pl.BlockSpec((N_POS // 128, 128), lambda i: (0, 0)),
        ],
        out_shape=jax.ShapeDtypeStruct((1, 1), jnp.float32),
        out_specs=pl.BlockSpec(memory_space=pltpu.SMEM),
        scratch_shapes=[pltpu.SMEM((2,), jnp.float32)],
    )(sp, mask0.reshape(ROWS, 128), mask1.reshape(ROWS, 128),
      in_pos.reshape(N_POS // 128, 128))
    return out[0, 0]
